# Initial kernel scaffold; baseline (speedup 1.0000x reference)
import jax
import jax.numpy as jnp
from jax.experimental import pallas as pl


def kernel(x, edge_index, W1, b1, W2, b2):
    raise NotImplementedError("write your pallas kernel here")


# speedup vs baseline: 9.5689x; 9.5689x over previous
import functools

import jax
import jax.numpy as jnp
from jax import lax
from jax.experimental import pallas as pl
from jax.experimental.pallas import tpu as pltpu
from jax.experimental.pallas import tpu_sc as plsc

_N = 10000
_NP = 10240
_E = 320000
_F = 128
_H = 64
_C = 47
_CP = 48
_ALPHA = 0.1
_KHALF = 5

_NC = 2
_NS = 16
_NW = _NC * _NS
_BLK = 128
_NBLK = 80
_EPT = _NBLK * _BLK
_EP = _EPT * _NW
_RPT = _NP // _NS

_ROW_BLK = 1280
_GRID = _NP // _ROW_BLK

_mesh = plsc.VectorSubcoreMesh(core_axis_name="c", subcore_axis_name="s")
_f32 = jnp.float32
_sc_params = pltpu.CompilerParams(use_tc_tiling_on_sc=False)


def _mlp_body(x_ref, w1_ref, b1_ref, w2_ref, b2_ref, o_ref):
    dn = (((1,), (0,)), ((), ()))
    h1 = lax.dot_general(x_ref[...], w1_ref[...], dn,
                         precision=lax.Precision.HIGHEST,
                         preferred_element_type=_f32)
    h1 = jnp.maximum(h1 + b1_ref[...], 0.0)
    o_ref[...] = lax.dot_general(h1, w2_ref[...], dn,
                                 precision=lax.Precision.HIGHEST,
                                 preferred_element_type=_f32) + b2_ref[...]


def _mlp(xp, W1, b1r, W2p, b2r):
    return pl.pallas_call(
        _mlp_body,
        grid=(_GRID,),
        in_specs=[
            pl.BlockSpec((_ROW_BLK, _F), lambda i: (i, 0)),
            pl.BlockSpec((_F, _H), lambda i: (0, 0)),
            pl.BlockSpec((1, _H), lambda i: (0, 0)),
            pl.BlockSpec((_H, _CP), lambda i: (0, 0)),
            pl.BlockSpec((1, _CP), lambda i: (0, 0)),
        ],
        out_specs=pl.BlockSpec((_ROW_BLK, _CP), lambda i: (i, 0)),
        out_shape=jax.ShapeDtypeStruct((_NP, _CP), _f32),
    )(xp, W1, b1r, W2p, b2r)


@functools.partial(
    pl.kernel,
    out_type=jax.ShapeDtypeStruct((_NC, _NP, 16), _f32),
    mesh=_mesh,
    scratch_types=[
        pltpu.VMEM_SHARED((_NP, 16), _f32),
        pltpu.VMEM((_NBLK, _BLK), jnp.int32),
        pltpu.VMEM((_BLK, 16), _f32),
    ],
    compiler_params=_sc_params,
)
def _deg_kernel(dst_hbm, zeros_hbm, ones_hbm, out_hbm, acc_sh, dst_v, ones_v):
    cid = lax.axis_index("c")
    sid = lax.axis_index("s")
    r0 = sid * _RPT
    pltpu.sync_copy(zeros_hbm, acc_sh.at[pl.ds(r0, _RPT)])
    pltpu.sync_copy(ones_hbm, ones_v)
    base = (cid * _NS + sid) * _NBLK
    pltpu.sync_copy(dst_hbm.at[pl.ds(base, _NBLK)], dst_v)
    plsc.subcore_barrier()

    @pl.loop(0, _NBLK)
    def _(j):
        pltpu.sync_copy(ones_v, acc_sh.at[dst_v.at[j]], add=True)

    plsc.subcore_barrier()
    pltpu.sync_copy(acc_sh.at[pl.ds(r0, _RPT)],
                    out_hbm.at[cid, pl.ds(r0, _RPT)])


@functools.partial(
    pl.kernel,
    out_type=jax.ShapeDtypeStruct((_NC, _NP, _CP), _f32),
    mesh=_mesh,
    scratch_types=[
        pltpu.VMEM_SHARED((_NP, _CP), _f32),
        pltpu.VMEM((_NBLK, _BLK), jnp.int32),
        pltpu.VMEM((_NBLK, _BLK), jnp.int32),
        pltpu.VMEM((_BLK, _CP), _f32),
        pltpu.SemaphoreType.DMA,
    ],
    compiler_params=_sc_params,
)
def _hop_kernel(v_hbm, src_hbm, dst_hbm, zeros_hbm, out_hbm,
                acc_sh, src_v, dst_v, rows_v, sem):
    cid = lax.axis_index("c")
    sid = lax.axis_index("s")
    r0 = sid * _RPT
    pltpu.sync_copy(zeros_hbm, acc_sh.at[pl.ds(r0, _RPT)])
    base = (cid * _NS + sid) * _NBLK
    pltpu.sync_copy(src_hbm.at[pl.ds(base, _NBLK)], src_v)
    pltpu.sync_copy(dst_hbm.at[pl.ds(base, _NBLK)], dst_v)
    plsc.subcore_barrier()

    @pl.loop(0, _NBLK)
    def _(j):
        pltpu.async_copy(v_hbm.at[src_v.at[j]], rows_v, sem).wait()
        pltpu.sync_copy(rows_v, acc_sh.at[dst_v.at[j]], add=True)

    plsc.subcore_barrier()
    pltpu.sync_copy(acc_sh.at[pl.ds(r0, _RPT)],
                    out_hbm.at[cid, pl.ds(r0, _RPT)])


def _prep_body(parts_ref, h_ref, d2_ref, v0_ref, acc0_ref):
    p = parts_ref[...]
    deg = p[0, :, 0:1] + p[1, :, 0:1] + 1.0
    d2_ref[...] = 1.0 / deg
    v0 = lax.rsqrt(deg) * h_ref[...]
    v0_ref[...] = v0
    acc0_ref[...] = _ALPHA * v0


def _prep(parts, h):
    return pl.pallas_call(
        _prep_body,
        grid=(_GRID,),
        in_specs=[
            pl.BlockSpec((_NC, _ROW_BLK, 16), lambda i: (0, i, 0)),
            pl.BlockSpec((_ROW_BLK, _CP), lambda i: (i, 0)),
        ],
        out_specs=[
            pl.BlockSpec((_ROW_BLK, 1), lambda i: (i, 0)),
            pl.BlockSpec((_ROW_BLK, _CP), lambda i: (i, 0)),
            pl.BlockSpec((_ROW_BLK, _CP), lambda i: (i, 0)),
        ],
        out_shape=[
            jax.ShapeDtypeStruct((_NP, 1), _f32),
            jax.ShapeDtypeStruct((_NP, _CP), _f32),
            jax.ShapeDtypeStruct((_NP, _CP), _f32),
        ],
    )(parts, h)


def _combine_body(s_ref, v_ref, d2_ref, vn_ref):
    s = s_ref[...]
    vn_ref[...] = d2_ref[...] * (s[0] + s[1] + v_ref[...])


def _combine(s, v, d2):
    return pl.pallas_call(
        _combine_body,
        grid=(_GRID,),
        in_specs=[
            pl.BlockSpec((_NC, _ROW_BLK, _CP), lambda i: (0, i, 0)),
            pl.BlockSpec((_ROW_BLK, _CP), lambda i: (i, 0)),
            pl.BlockSpec((_ROW_BLK, 1), lambda i: (i, 0)),
        ],
        out_specs=pl.BlockSpec((_ROW_BLK, _CP), lambda i: (i, 0)),
        out_shape=jax.ShapeDtypeStruct((_NP, _CP), _f32),
    )(s, v, d2)


def _combine_acc_body(c, s_ref, v_ref, d2_ref, acc_ref, vn_ref, accn_ref):
    s = s_ref[...]
    vn = d2_ref[...] * (s[0] + s[1] + v_ref[...])
    vn_ref[...] = vn
    accn_ref[...] = acc_ref[...] + c * vn


def _combine_acc(s, v, d2, acc, c):
    return pl.pallas_call(
        functools.partial(_combine_acc_body, c),
        grid=(_GRID,),
        in_specs=[
            pl.BlockSpec((_NC, _ROW_BLK, _CP), lambda i: (0, i, 0)),
            pl.BlockSpec((_ROW_BLK, _CP), lambda i: (i, 0)),
            pl.BlockSpec((_ROW_BLK, 1), lambda i: (i, 0)),
            pl.BlockSpec((_ROW_BLK, _CP), lambda i: (i, 0)),
        ],
        out_specs=[
            pl.BlockSpec((_ROW_BLK, _CP), lambda i: (i, 0)),
            pl.BlockSpec((_ROW_BLK, _CP), lambda i: (i, 0)),
        ],
        out_shape=[
            jax.ShapeDtypeStruct((_NP, _CP), _f32),
            jax.ShapeDtypeStruct((_NP, _CP), _f32),
        ],
    )(s, v, d2, acc)


def _lsm_body(acc_ref, d2_ref, o_ref):
    t = lax.rsqrt(d2_ref[...]) * acc_ref[...]
    col = lax.broadcasted_iota(jnp.int32, t.shape, 1)
    valid = col < _C
    m = jnp.max(jnp.where(valid, t, -1e30), axis=1, keepdims=True)
    e = jnp.where(valid, jnp.exp(t - m), 0.0)
    o_ref[...] = (t - m) - jnp.log(jnp.sum(e, axis=1, keepdims=True))


def _lsm(acc, d2):
    return pl.pallas_call(
        _lsm_body,
        grid=(_GRID,),
        in_specs=[
            pl.BlockSpec((_ROW_BLK, _CP), lambda i: (i, 0)),
            pl.BlockSpec((_ROW_BLK, 1), lambda i: (i, 0)),
        ],
        out_specs=pl.BlockSpec((_ROW_BLK, _CP), lambda i: (i, 0)),
        out_shape=jax.ShapeDtypeStruct((_NP, _CP), _f32),
    )(acc, d2)


def kernel(x, edge_index, W1, b1, W2, b2):
    xp = jnp.zeros((_NP, _F), _f32).at[:_N].set(x)
    W2p = jnp.pad(W2, ((0, 0), (0, _CP - _C)))
    b1r = b1.reshape(1, _H)
    b2r = jnp.pad(b2, (0, _CP - _C)).reshape(1, _CP)

    pad = jnp.full((_EP - _E,), _N, jnp.int32)
    src2 = jnp.concatenate([edge_index[0].astype(jnp.int32), pad]
                           ).reshape(_NW * _NBLK, _BLK)
    dst2 = jnp.concatenate([edge_index[1].astype(jnp.int32), pad]
                           ).reshape(_NW * _NBLK, _BLK)
    zeros16 = jnp.zeros((_RPT, 16), _f32)
    ones16 = jnp.ones((_BLK, 16), _f32)
    zeros48 = jnp.zeros((_RPT, _CP), _f32)

    h = _mlp(xp, W1, b1r, W2p, b2r)
    parts = _deg_kernel(dst2, zeros16, ones16)
    d2, v, acc = _prep(parts, h)

    coef = [_ALPHA * (1.0 - _ALPHA) ** i for i in range(_KHALF + 1)]
    coef[_KHALF] = (1.0 - _ALPHA) ** _KHALF
    for i in range(1, _KHALF + 1):
        s = _hop_kernel(v, src2, dst2, zeros48)
        v = _combine(s, v, d2)
        s = _hop_kernel(v, src2, dst2, zeros48)
        v, acc = _combine_acc(s, v, d2, acc, coef[i])

    out = _lsm(acc, d2)
    return out[:_N, :_C]
